# baseline (speedup 1.0000x reference)
import functools
import math

import jax
import jax.numpy as jnp
from jax import lax
from jax.experimental import pallas as pl
from jax.experimental.pallas import tpu as pltpu
from jax.experimental.pallas import tpu_sc as plsc

N_IDX = 2048
N_BATCH = 6
D = 256
TAB = N_BATCH * 4096

_NC = 2
_NS = 16
_NW = _NC * _NS
_PER_TILE = N_IDX // _NW
_ROWS = N_BATCH * D // _NW


def _sc_mesh():
    return plsc.VectorSubcoreMesh(core_axis_name="c", subcore_axis_name="s",
                                  num_cores=_NC, num_subcores=_NS)


def _base_body(p1_ref, p3_ref, base_ref, mul_ref):
    mt = (0.975 * p3_ref[0]).T
    mul_ref[0] = mt
    base_ref[0] = p1_ref[0].T + mt


def _base_call(p1, p3):
    spec = pl.BlockSpec((1, D, D), lambda b: (b, 0, 0))
    return pl.pallas_call(
        _base_body,
        grid=(N_BATCH,),
        in_specs=[spec, spec],
        out_specs=[spec, spec],
        out_shape=[jax.ShapeDtypeStruct((N_BATCH, D, D), jnp.float32)] * 2,
    )(p1, p3)


def _copy_body(p1_ref, p3_ref, p1c_ref, p3c_ref):
    p1c_ref[0] = p1_ref[0]
    p3c_ref[0] = p3_ref[0]


def _copy_call(p1, p3):
    spec = pl.BlockSpec((1, D, D), lambda b: (b, 0, 0))
    return pl.pallas_call(
        _copy_body,
        grid=(N_BATCH,),
        in_specs=[spec, spec],
        out_specs=[spec, spec],
        out_shape=[jax.ShapeDtypeStruct((N_BATCH, D, D), jnp.float32)] * 2,
    )(p1, p3)


def _gather_body(p5h, p6h, p7h, p8h, tabs_hbm,
                 v0_hbm, v1_hbm, v2_hbm, tpm_hbm,
                 b5, b6, b7, b8, g0_v, g1_v, g2_v, tpm_v,
                 r0, r1, r2, s0, s1, s2):
    wid = lax.axis_index("s") * _NC + lax.axis_index("c")
    base = wid * _PER_TILE
    pltpu.sync_copy(p5h.at[pl.ds(base, _PER_TILE)], b5)
    pltpu.sync_copy(p6h.at[pl.ds(base, _PER_TILE)], b6)
    pltpu.sync_copy(p7h.at[pl.ds(base, _PER_TILE)], b7)
    pltpu.sync_copy(p8h.at[pl.ds(base, _PER_TILE)], b8)
    lane = lax.broadcasted_iota(jnp.int32, (16,), 0)
    for j in range(_PER_TILE // 16):
        sl = pl.ds(j * 16, 16)
        b = b5[sl]
        e = b6[sl]
        r = b7[sl]
        c = b8[sl]
        g = b * 4096 + e
        g0_v[sl] = g
        g1_v[sl] = g + TAB
        g2_v[sl] = g + 2 * TAB
        tp = b * 65536 + c * 256 + r
        dead = lane < 0
        for k in range(1, 16):
            idx = jnp.minimum(lane + k, 15)
            sh = tp.at[idx].get(mode="promise_in_bounds")
            dead = dead | ((sh == tp) & (lane + k < 16))
        tpm_v[sl] = jnp.where(dead, -1, tp)
    c0 = pltpu.async_copy(tabs_hbm.at[g0_v], r0, s0)
    c1 = pltpu.async_copy(tabs_hbm.at[g1_v], r1, s1)
    c2 = pltpu.async_copy(tabs_hbm.at[g2_v], r2, s2)
    c0.wait()
    c1.wait()
    c2.wait()
    pltpu.sync_copy(r0, v0_hbm.at[pl.ds(base, _PER_TILE)])
    pltpu.sync_copy(r1, v1_hbm.at[pl.ds(base, _PER_TILE)])
    pltpu.sync_copy(r2, v2_hbm.at[pl.ds(base, _PER_TILE)])
    pltpu.sync_copy(tpm_v, tpm_hbm.at[pl.ds(base, _PER_TILE)])


@functools.lru_cache(maxsize=None)
def _gather_call_fn():
    return pl.kernel(
        _gather_body,
        out_type=[jax.ShapeDtypeStruct((N_IDX,), jnp.float32)] * 3
        + [jax.ShapeDtypeStruct((N_IDX,), jnp.int32)],
        mesh=_sc_mesh(),
        compiler_params=pltpu.CompilerParams(needs_layout_passes=False),
        scratch_types=[
            pltpu.VMEM((_PER_TILE,), jnp.int32),
            pltpu.VMEM((_PER_TILE,), jnp.int32),
            pltpu.VMEM((_PER_TILE,), jnp.int32),
            pltpu.VMEM((_PER_TILE,), jnp.int32),
            pltpu.VMEM((_PER_TILE,), jnp.int32),
            pltpu.VMEM((_PER_TILE,), jnp.int32),
            pltpu.VMEM((_PER_TILE,), jnp.int32),
            pltpu.VMEM((_PER_TILE,), jnp.int32),
            pltpu.VMEM((_PER_TILE,), jnp.float32),
            pltpu.VMEM((_PER_TILE,), jnp.float32),
            pltpu.VMEM((_PER_TILE,), jnp.float32),
            pltpu.SemaphoreType.DMA,
            pltpu.SemaphoreType.DMA,
            pltpu.SemaphoreType.DMA,
        ],
    )


def _gather_call(p5, p6, p7, p8, tabs):
    return _gather_call_fn()(p5, p6, p7, p8, tabs)


def _scat_body(baseT_hbm, mulT_hbm, tpm_hbm, v0_hbm, v1_hbm, v2_hbm,
               o24_hbm, o20_hbm, o16_hbm,
               buf, mbuf, tpm_v, v0, v1, v2, hri,
               sa, sm, so):
    wid = lax.axis_index("s") * _NC + lax.axis_index("c")
    row0 = wid * _ROWS
    handles = []
    for s in range(_ROWS // 16):
        r = row0 + s * 16
        bb = lax.div(r, D)
        rr = lax.rem(r, D)
        dsl = pl.ds(s * 16, 16)
        handles.append(pltpu.async_copy(baseT_hbm.at[bb, pl.ds(rr, 16), :],
                                        buf.at[dsl, :], sa))
        handles.append(pltpu.async_copy(mulT_hbm.at[bb, pl.ds(rr, 16), :],
                                        mbuf.at[dsl, :], sm))
    pltpu.sync_copy(tpm_hbm, tpm_v)
    pltpu.sync_copy(v0_hbm, v0)
    pltpu.sync_copy(v1_hbm, v1)
    pltpu.sync_copy(v2_hbm, v2)
    lane = lax.broadcasted_iota(jnp.int32, (16,), 0)

    def cbody(j, cnt):
        sl = pl.ds(j * 16, 16)
        tp = tpm_v[sl]
        trow = lax.shift_right_arithmetic(tp, 8)
        m = (trow >= row0) & (trow < row0 + _ROWS)
        packed = ((trow - row0) * 524288 + lax.bitwise_and(tp, 255) * 2048
                  + j * 16 + lane)
        plsc.store_compressed(hri.at[pl.ds(cnt, 16)], packed, mask=m)
        return cnt + jnp.max(plsc.all_reduce_population_count(m))

    cnt = lax.fori_loop(0, N_IDX // 16, cbody, 0)
    ntr = lax.div(cnt + 15, 16)
    for h in handles:
        h.wait()

    def stage(vref, dst_hbm, sem):
        def sbody(t, _):
            sl = pl.ds(t * 16, 16)
            valid = (t * 16 + lane) < cnt
            packed = jnp.where(valid, hri[sl], 0)
            ri = lax.shift_right_logical(packed, 19)
            ci = lax.bitwise_and(lax.shift_right_logical(packed, 11), 255)
            idv = lax.bitwise_and(packed, 2047)
            key = jnp.where(valid, ri * D + ci, -1)
            dead = lane < 0
            for k in range(1, 16):
                idx = jnp.minimum(lane + k, 15)
                sh = key.at[idx].get(mode="promise_in_bounds")
                dead = dead | ((sh == key) & (lane + k < 16))
            m = valid & (~dead)
            mu = plsc.load_gather(mbuf, [ri, ci], mask=m)
            w = plsc.load_gather(vref, [idv], mask=m) + mu
            plsc.store_scatter(buf, [ri, ci], w, mask=m)
            return 0

        lax.fori_loop(0, ntr, sbody, 0)
        oh = []
        for s in range(_ROWS // 16):
            r = row0 + s * 16
            bb = lax.div(r, D)
            rr = lax.rem(r, D)
            dsl = pl.ds(s * 16, 16)
            oh.append(pltpu.async_copy(buf.at[dsl, :],
                                       dst_hbm.at[bb, pl.ds(rr, 16), :], sem))
        for h in oh:
            h.wait()

    stage(v0, o24_hbm, so)
    stage(v1, o20_hbm, so)
    stage(v2, o16_hbm, so)


@functools.lru_cache(maxsize=None)
def _scat_call_fn():
    return pl.kernel(
        _scat_body,
        out_type=[jax.ShapeDtypeStruct((N_BATCH, D, D), jnp.float32)] * 3,
        mesh=_sc_mesh(),
        compiler_params=pltpu.CompilerParams(needs_layout_passes=False,
                                             use_tc_tiling_on_sc=True),
        scratch_types=[
            pltpu.VMEM((_ROWS, D), jnp.float32),
            pltpu.VMEM((_ROWS, D), jnp.float32),
            pltpu.VMEM((N_IDX,), jnp.int32),
            pltpu.VMEM((N_IDX,), jnp.float32),
            pltpu.VMEM((N_IDX,), jnp.float32),
            pltpu.VMEM((N_IDX,), jnp.float32),
            pltpu.VMEM((N_IDX + 16,), jnp.int32),
            pltpu.SemaphoreType.DMA,
            pltpu.SemaphoreType.DMA,
            pltpu.SemaphoreType.DMA,
        ],
    )


def _scat_call(baseT, mulT, tpm, v0, v1, v2):
    return _scat_call_fn()(baseT, mulT, tpm, v0, v1, v2)


def _bmm_body(a_ref, t0_ref, t1_ref, t2_ref,
              s9, s10, s12, s13, s15, s16,
              div1_ref, sub1_ref, sub2_ref):
    tc = lax.broadcasted_iota(jnp.int32, (D, 1), 0).astype(jnp.float32) * (
        2.0 * math.pi)
    sub0c = jnp.sin(tc * s9[0, 0] + s10[0, 0]) ** 2 * 0.1 + 1.0 - 0.05
    sub1c = jnp.sin(tc * s12[0, 0] + s13[0, 0]) ** 2 * 0.1 + 1.0 - 0.05
    tr = lax.broadcasted_iota(jnp.int32, (1, D), 1).astype(jnp.float32) * (
        2.0 * math.pi)
    sub1r = jnp.sin(tr * s12[0, 0] + s13[0, 0]) ** 2 * 0.1 + 1.0 - 0.05
    sub2r = jnp.sin(tr * s15[0, 0] + s16[0, 0]) ** 2 * 0.1 + 1.0 - 0.05
    rs0 = 1.0 / sub0c
    rs2r = 1.0 / sub2r
    for b in range(N_BATCH):
        a = a_ref[:, b, :]
        T0 = t0_ref[b]
        T1 = t1_ref[b]
        T2 = t2_ref[b]
        m0 = lax.dot_general(T0, a, (((1,), (1,)), ((), ())),
                             preferred_element_type=jnp.float32)
        d0 = m0 * rs0
        m1 = lax.dot_general(T1, d0, (((1,), (0,)), ((), ())),
                             preferred_element_type=jnp.float32)
        t1m = m1 * sub1c
        m2t = lax.dot_general(t1m, T2, (((0,), (1,)), ((), ())),
                              preferred_element_type=jnp.float32)
        div1_ref[:, b, :] = m2t * rs2r
    sub1_ref[...] = sub1r
    sub2_ref[...] = sub2r


def _bmm_call(p2, o24, o20, o16, p9, p10, p12, p13, p15, p16):
    scal = lambda x: x.reshape(1, 1).astype(jnp.float32)
    div1, sub1, sub2 = pl.pallas_call(
        _bmm_body,
        out_shape=[jax.ShapeDtypeStruct((12, N_BATCH, D), jnp.float32),
                   jax.ShapeDtypeStruct((1, D), jnp.float32),
                   jax.ShapeDtypeStruct((1, D), jnp.float32)],
    )(p2, o24, o20, o16, scal(p9), scal(p10), scal(p12), scal(p13),
      scal(p15), scal(p16))
    return div1, sub1.reshape(D), sub2.reshape(D)


def kernel(primals_1, primals_2, primals_3, primals_4, primals_5, primals_6,
           primals_7, primals_8, primals_9, primals_10, primals_11,
           primals_12, primals_13, primals_14, primals_15, primals_16):
    baseT, mulT = _base_call(primals_1, primals_3)
    tabs = jnp.concatenate([primals_4, primals_11, primals_14]).reshape(-1)
    v0, v1, v2, tpm = _gather_call(primals_5, primals_6, primals_7, primals_8,
                                   tabs)
    o24, o20, o16 = _scat_call(baseT, mulT, tpm, v0, v1, v2)
    p1c, p3c = _copy_call(primals_1, primals_3)
    div1, sub1, sub2 = _bmm_call(primals_2, o24, o20, o16, primals_9,
                                 primals_10, primals_12, primals_13,
                                 primals_15, primals_16)
    return (div1, p3c, p1c, primals_9, primals_10, sub1, sub2,
            o16, o20, o24)


# speedup vs baseline: 1.0111x; 1.0111x over previous
import functools
import math

import jax
import jax.numpy as jnp
from jax import lax
from jax.experimental import pallas as pl
from jax.experimental.pallas import tpu as pltpu
from jax.experimental.pallas import tpu_sc as plsc

N_IDX = 2048
N_BATCH = 6
D = 256
TAB = N_BATCH * 4096

_NC = 2
_NS = 16
_NW = _NC * _NS
_PER_TILE = N_IDX // _NW
_ROWS = N_BATCH * D // _NW


def _sc_mesh():
    return plsc.VectorSubcoreMesh(core_axis_name="c", subcore_axis_name="s",
                                  num_cores=_NC, num_subcores=_NS)


def _base_body(p1_ref, p3_ref, base_ref, mul_ref):
    mt = (0.975 * p3_ref[0]).T
    mul_ref[0] = mt
    base_ref[0] = p1_ref[0].T + mt


def _base_call(p1, p3):
    spec = pl.BlockSpec((1, D, D), lambda b: (b, 0, 0))
    return pl.pallas_call(
        _base_body,
        grid=(N_BATCH,),
        in_specs=[spec, spec],
        out_specs=[spec, spec],
        out_shape=[jax.ShapeDtypeStruct((N_BATCH, D, D), jnp.float32)] * 2,
    )(p1, p3)


def _copy_body(p1_ref, p3_ref, p1c_ref, p3c_ref):
    p1c_ref[0] = p1_ref[0]
    p3c_ref[0] = p3_ref[0]


def _copy_call(p1, p3):
    spec = pl.BlockSpec((1, D, D), lambda b: (b, 0, 0))
    return pl.pallas_call(
        _copy_body,
        grid=(N_BATCH,),
        in_specs=[spec, spec],
        out_specs=[spec, spec],
        out_shape=[jax.ShapeDtypeStruct((N_BATCH, D, D), jnp.float32)] * 2,
    )(p1, p3)


def _gather_body(p5h, p6h, p7h, p8h, tabs_hbm,
                 v0_hbm, v1_hbm, v2_hbm, tpm_hbm,
                 b5, b6, b7, b8, g0_v, g1_v, g2_v, tpm_v,
                 r0, r1, r2, s0, s1, s2):
    wid = lax.axis_index("s") * _NC + lax.axis_index("c")
    base = wid * _PER_TILE
    pltpu.sync_copy(p5h.at[pl.ds(base, _PER_TILE)], b5)
    pltpu.sync_copy(p6h.at[pl.ds(base, _PER_TILE)], b6)
    pltpu.sync_copy(p7h.at[pl.ds(base, _PER_TILE)], b7)
    pltpu.sync_copy(p8h.at[pl.ds(base, _PER_TILE)], b8)
    for j in range(_PER_TILE // 16):
        sl = pl.ds(j * 16, 16)
        b = b5[sl]
        e = b6[sl]
        r = b7[sl]
        c = b8[sl]
        g = b * 4096 + e
        g0_v[sl] = g
        g1_v[sl] = g + TAB
        g2_v[sl] = g + 2 * TAB
        tpm_v[sl] = b * 65536 + c * 256 + r
    c0 = pltpu.async_copy(tabs_hbm.at[g0_v], r0, s0)
    c1 = pltpu.async_copy(tabs_hbm.at[g1_v], r1, s1)
    c2 = pltpu.async_copy(tabs_hbm.at[g2_v], r2, s2)
    c0.wait()
    c1.wait()
    c2.wait()
    pltpu.sync_copy(r0, v0_hbm.at[pl.ds(base, _PER_TILE)])
    pltpu.sync_copy(r1, v1_hbm.at[pl.ds(base, _PER_TILE)])
    pltpu.sync_copy(r2, v2_hbm.at[pl.ds(base, _PER_TILE)])
    pltpu.sync_copy(tpm_v, tpm_hbm.at[pl.ds(base, _PER_TILE)])


@functools.lru_cache(maxsize=None)
def _gather_call_fn():
    return pl.kernel(
        _gather_body,
        out_type=[jax.ShapeDtypeStruct((N_IDX,), jnp.float32)] * 3
        + [jax.ShapeDtypeStruct((N_IDX,), jnp.int32)],
        mesh=_sc_mesh(),
        compiler_params=pltpu.CompilerParams(needs_layout_passes=False),
        scratch_types=[
            pltpu.VMEM((_PER_TILE,), jnp.int32),
            pltpu.VMEM((_PER_TILE,), jnp.int32),
            pltpu.VMEM((_PER_TILE,), jnp.int32),
            pltpu.VMEM((_PER_TILE,), jnp.int32),
            pltpu.VMEM((_PER_TILE,), jnp.int32),
            pltpu.VMEM((_PER_TILE,), jnp.int32),
            pltpu.VMEM((_PER_TILE,), jnp.int32),
            pltpu.VMEM((_PER_TILE,), jnp.int32),
            pltpu.VMEM((_PER_TILE,), jnp.float32),
            pltpu.VMEM((_PER_TILE,), jnp.float32),
            pltpu.VMEM((_PER_TILE,), jnp.float32),
            pltpu.SemaphoreType.DMA,
            pltpu.SemaphoreType.DMA,
            pltpu.SemaphoreType.DMA,
        ],
    )


def _gather_call(p5, p6, p7, p8, tabs):
    return _gather_call_fn()(p5, p6, p7, p8, tabs)


def _scat_body(baseT_hbm, mulT_hbm, tpm_hbm, v0_hbm, v1_hbm, v2_hbm,
               o24_hbm, o20_hbm, o16_hbm,
               buf, mbuf, tpm_v, v0, v1, v2, hri,
               sa, sm, so):
    wid = lax.axis_index("s") * _NC + lax.axis_index("c")
    row0 = wid * _ROWS
    handles = []
    for s in range(_ROWS // 16):
        r = row0 + s * 16
        bb = lax.div(r, D)
        rr = lax.rem(r, D)
        dsl = pl.ds(s * 16, 16)
        handles.append(pltpu.async_copy(baseT_hbm.at[bb, pl.ds(rr, 16), :],
                                        buf.at[dsl, :], sa))
        handles.append(pltpu.async_copy(mulT_hbm.at[bb, pl.ds(rr, 16), :],
                                        mbuf.at[dsl, :], sm))
    pltpu.sync_copy(tpm_hbm, tpm_v)
    pltpu.sync_copy(v0_hbm, v0)
    pltpu.sync_copy(v1_hbm, v1)
    pltpu.sync_copy(v2_hbm, v2)
    lane = lax.broadcasted_iota(jnp.int32, (16,), 0)

    def cbody(j, cnt):
        sl = pl.ds(j * 16, 16)
        tp = tpm_v[sl]
        trow = lax.shift_right_arithmetic(tp, 8)
        m = (trow >= row0) & (trow < row0 + _ROWS)
        packed = ((trow - row0) * 524288 + lax.bitwise_and(tp, 255) * 2048
                  + j * 16 + lane)
        plsc.store_compressed(hri.at[pl.ds(cnt, 16)], packed, mask=m)
        return cnt + jnp.max(plsc.all_reduce_population_count(m))

    cnt = lax.fori_loop(0, N_IDX // 16, cbody, 0)
    ntr = lax.div(cnt + 15, 16)
    for h in handles:
        h.wait()

    def stage(vref, dst_hbm, sem):
        def sbody(t, _):
            sl = pl.ds(t * 16, 16)
            valid = (t * 16 + lane) < cnt
            packed = jnp.where(valid, hri[sl], 0)
            ri = lax.shift_right_logical(packed, 19)
            ci = lax.bitwise_and(lax.shift_right_logical(packed, 11), 255)
            idv = lax.bitwise_and(packed, 2047)
            key = jnp.where(valid, ri * D + ci, -1)
            dead = lane < 0
            for k in range(1, 16):
                idx = jnp.minimum(lane + k, 15)
                sh = key.at[idx].get(mode="promise_in_bounds")
                dead = dead | ((sh == key) & (lane + k < 16))
            m = valid & (~dead)
            mu = plsc.load_gather(mbuf, [ri, ci], mask=m)
            w = plsc.load_gather(vref, [idv], mask=m) + mu
            plsc.store_scatter(buf, [ri, ci], w, mask=m)
            return 0

        lax.fori_loop(0, ntr, sbody, 0)
        oh = []
        for s in range(_ROWS // 16):
            r = row0 + s * 16
            bb = lax.div(r, D)
            rr = lax.rem(r, D)
            dsl = pl.ds(s * 16, 16)
            oh.append(pltpu.async_copy(buf.at[dsl, :],
                                       dst_hbm.at[bb, pl.ds(rr, 16), :], sem))
        for h in oh:
            h.wait()

    stage(v0, o24_hbm, so)
    stage(v1, o20_hbm, so)
    stage(v2, o16_hbm, so)


@functools.lru_cache(maxsize=None)
def _scat_call_fn():
    return pl.kernel(
        _scat_body,
        out_type=[jax.ShapeDtypeStruct((N_BATCH, D, D), jnp.float32)] * 3,
        mesh=_sc_mesh(),
        compiler_params=pltpu.CompilerParams(needs_layout_passes=False,
                                             use_tc_tiling_on_sc=True),
        scratch_types=[
            pltpu.VMEM((_ROWS, D), jnp.float32),
            pltpu.VMEM((_ROWS, D), jnp.float32),
            pltpu.VMEM((N_IDX,), jnp.int32),
            pltpu.VMEM((N_IDX,), jnp.float32),
            pltpu.VMEM((N_IDX,), jnp.float32),
            pltpu.VMEM((N_IDX,), jnp.float32),
            pltpu.VMEM((N_IDX + 16,), jnp.int32),
            pltpu.SemaphoreType.DMA,
            pltpu.SemaphoreType.DMA,
            pltpu.SemaphoreType.DMA,
        ],
    )


def _scat_call(baseT, mulT, tpm, v0, v1, v2):
    return _scat_call_fn()(baseT, mulT, tpm, v0, v1, v2)


def _bmm_body(a_ref, t0_ref, t1_ref, t2_ref,
              s9, s10, s12, s13, s15, s16,
              div1_ref, sub1_ref, sub2_ref):
    tc = lax.broadcasted_iota(jnp.int32, (D, 1), 0).astype(jnp.float32) * (
        2.0 * math.pi)
    sub0c = jnp.sin(tc * s9[0, 0] + s10[0, 0]) ** 2 * 0.1 + 1.0 - 0.05
    sub1c = jnp.sin(tc * s12[0, 0] + s13[0, 0]) ** 2 * 0.1 + 1.0 - 0.05
    tr = lax.broadcasted_iota(jnp.int32, (1, D), 1).astype(jnp.float32) * (
        2.0 * math.pi)
    sub1r = jnp.sin(tr * s12[0, 0] + s13[0, 0]) ** 2 * 0.1 + 1.0 - 0.05
    sub2r = jnp.sin(tr * s15[0, 0] + s16[0, 0]) ** 2 * 0.1 + 1.0 - 0.05
    rs0 = 1.0 / sub0c
    rs2r = 1.0 / sub2r
    for b in range(N_BATCH):
        a = a_ref[:, b, :]
        T0 = t0_ref[b]
        T1 = t1_ref[b]
        T2 = t2_ref[b]
        m0 = lax.dot_general(T0, a, (((1,), (1,)), ((), ())),
                             preferred_element_type=jnp.float32)
        d0 = m0 * rs0
        m1 = lax.dot_general(T1, d0, (((1,), (0,)), ((), ())),
                             preferred_element_type=jnp.float32)
        t1m = m1 * sub1c
        m2t = lax.dot_general(t1m, T2, (((0,), (1,)), ((), ())),
                              preferred_element_type=jnp.float32)
        div1_ref[:, b, :] = m2t * rs2r
    sub1_ref[...] = sub1r
    sub2_ref[...] = sub2r


def _bmm_call(p2, o24, o20, o16, p9, p10, p12, p13, p15, p16):
    scal = lambda x: x.reshape(1, 1).astype(jnp.float32)
    div1, sub1, sub2 = pl.pallas_call(
        _bmm_body,
        out_shape=[jax.ShapeDtypeStruct((12, N_BATCH, D), jnp.float32),
                   jax.ShapeDtypeStruct((1, D), jnp.float32),
                   jax.ShapeDtypeStruct((1, D), jnp.float32)],
    )(p2, o24, o20, o16, scal(p9), scal(p10), scal(p12), scal(p13),
      scal(p15), scal(p16))
    return div1, sub1.reshape(D), sub2.reshape(D)


def kernel(primals_1, primals_2, primals_3, primals_4, primals_5, primals_6,
           primals_7, primals_8, primals_9, primals_10, primals_11,
           primals_12, primals_13, primals_14, primals_15, primals_16):
    baseT, mulT = _base_call(primals_1, primals_3)
    tabs = jnp.concatenate([primals_4, primals_11, primals_14]).reshape(-1)
    v0, v1, v2, tpm = _gather_call(primals_5, primals_6, primals_7, primals_8,
                                   tabs)
    o24, o20, o16 = _scat_call(baseT, mulT, tpm, v0, v1, v2)
    p1c, p3c = _copy_call(primals_1, primals_3)
    div1, sub1, sub2 = _bmm_call(primals_2, o24, o20, o16, primals_9,
                                 primals_10, primals_12, primals_13,
                                 primals_15, primals_16)
    return (div1, p3c, p1c, primals_9, primals_10, sub1, sub2,
            o16, o20, o24)
